# Initial kernel scaffold; baseline (speedup 1.0000x reference)
import jax
import jax.numpy as jnp
from jax.experimental import pallas as pl


def kernel(nodes, edges, r, senders, receivers, We1, be1, We2, be2, We3, be3, ge, bge, Wn1, bn1, Wn2, bn2, Wn3, bn3, gn, bgn, Wra, bra, Wsa, bsa):
    raise NotImplementedError("write your pallas kernel here")


# speedup vs baseline: 2.8331x; 2.8331x over previous
import dataclasses
import functools

import jax
import jax.numpy as jnp
from jax import lax
from jax.experimental import pallas as pl
from jax.experimental.pallas import tpu as pltpu
from jax.experimental.pallas import tpu_sc as plsc

N = 10000
E = 320000
D_NODE = 128
D_EDGE = 16
HID = 128
R_MAX = 6.0

_NC = 2
_NS = 16
_NW = _NC * _NS

_GCH = 80
_EPW = E // _NW

_SCC = 20000
_SNC = E // _SCC
_SNG = _SCC // 16

_NP = 10240
_BN = 400
_BNP = 512
_BE = 3200


def _mm(a, b):
    return jax.lax.dot_general(a, b, (((1,), (0,)), ((), ())),
                               preferred_element_type=jnp.float32)


def _mmT(a, b):
    return jax.lax.dot_general(a, b, (((0,), (0,)), ((), ())),
                               preferred_element_type=jnp.float32)


def _tables_kernel(nodes_ref, ws_ref, wr_ref, ns_ref, nr_ref):
    x = nodes_ref[...]
    ns_ref[...] = _mm(x, ws_ref[...])
    nr_ref[...] = _mm(x, wr_ref[...])


def _tables_call(nodes, We1s, We1r):
    return pl.pallas_call(
        _tables_kernel,
        grid=(N // _BN,),
        in_specs=[
            pl.BlockSpec((_BN, D_NODE), lambda i: (i, 0)),
            pl.BlockSpec((D_NODE, HID), lambda i: (0, 0)),
            pl.BlockSpec((D_NODE, HID), lambda i: (0, 0)),
        ],
        out_specs=[
            pl.BlockSpec((_BN, HID), lambda i: (i, 0)),
            pl.BlockSpec((_BN, HID), lambda i: (i, 0)),
        ],
        out_shape=[
            jax.ShapeDtypeStruct((N, HID), jnp.float32),
            jax.ShapeDtypeStruct((N, HID), jnp.float32),
        ],
    )(nodes, We1s, We1r)


def _gather_call(ns1, nr1, senders, receivers):
    mesh = plsc.VectorSubcoreMesh(core_axis_name="c", subcore_axis_name="s")

    @functools.partial(
        pl.kernel,
        mesh=mesh,
        out_type=[
            jax.ShapeDtypeStruct((E, HID), jnp.float32),
            jax.ShapeDtypeStruct((E, HID), jnp.float32),
        ],
        scratch_types=[
            pltpu.VMEM((_GCH,), jnp.int32),
            pltpu.VMEM((_GCH,), jnp.int32),
            pltpu.VMEM((_GCH, HID), jnp.float32),
            pltpu.VMEM((_GCH, HID), jnp.float32),
            pltpu.SemaphoreType.DMA,
            pltpu.SemaphoreType.DMA,
        ],
    )
    def k(ns1_h, nr1_h, s_h, r_h, gs_h, gr_h, is_v, ir_v, rs_v, rr_v,
          sem_g, sem_w):
        wid = lax.axis_index("s") * _NC + lax.axis_index("c")
        base = wid * _EPW

        @pl.loop(0, _EPW, step=_GCH)
        def _(off):
            sl = pl.ds(base + off, _GCH)
            pltpu.sync_copy(s_h.at[sl], is_v)
            pltpu.sync_copy(r_h.at[sl], ir_v)
            g1 = pltpu.async_copy(ns1_h.at[is_v], rs_v, sem_g)
            g2 = pltpu.async_copy(nr1_h.at[ir_v], rr_v, sem_g)
            g1.wait()
            g2.wait()
            w1 = pltpu.async_copy(rs_v, gs_h.at[sl], sem_w)
            w2 = pltpu.async_copy(rr_v, gr_h.at[sl], sem_w)
            w1.wait()
            w2.wait()

    return k(ns1, nr1, senders, receivers)


def _edge_kernel(e_ref, gs_ref, gr_ref, r_ref,
                 we1_ref, be1_ref, we2_ref, be2_ref, we3_ref, be3_ref,
                 ge_ref, bge_ref, wra_ref, bra_ref, wsa_ref, bsa_ref,
                 ne_ref, ws_ref, wr_ref):
    e = e_ref[...]
    r = r_ref[...]
    x = r * (1.0 / R_MAX)
    x2 = x * x
    x4 = x2 * x2
    x5 = x4 * x
    x6 = x5 * x
    envelope = 1.0 - 15.0 * x4 + 24.0 * x5 - 10.0 * x6
    cut = jnp.where(r < R_MAX, envelope, 0.0)
    ra = jax.nn.sigmoid(
        jnp.sum(e * wra_ref[...], axis=1, keepdims=True) + bra_ref[0, 0]) * cut
    sa = jax.nn.sigmoid(
        jnp.sum(e * wsa_ref[...], axis=1, keepdims=True) + bsa_ref[0, 0]) * cut
    h = _mm(e, we1_ref[...]) + gs_ref[...] + gr_ref[...] + be1_ref[...]
    h = h * jax.nn.sigmoid(h)
    h = _mm(h, we2_ref[...]) + be2_ref[...]
    h = h * jax.nn.sigmoid(h)
    h = _mm(h, we3_ref[...]) + be3_ref[...]
    mu = jnp.mean(h, axis=1, keepdims=True)
    d = h - mu
    var = jnp.mean(d * d, axis=1, keepdims=True)
    ue = d * jax.lax.rsqrt(var + 1e-5) * ge_ref[...] + bge_ref[...]
    ne_ref[...] = e + ue
    ws_ref[...] = (ue * sa).T
    wr_ref[...] = (ue * ra).T


def _edge_call(edges, gs, gr, rr, We1e, be1, We2, be2, We3, be3,
               ge, bge, wra, bra, wsa, bsa):
    rep2 = lambda shape: pl.BlockSpec(shape, lambda i: (0, 0))
    blk = lambda w: pl.BlockSpec((_BE, w), lambda i: (i, 0))
    return pl.pallas_call(
        _edge_kernel,
        grid=(E // _BE,),
        in_specs=[
            blk(D_EDGE), blk(HID), blk(HID), blk(1),
            rep2((D_EDGE, HID)), rep2((1, HID)),
            rep2((HID, HID)), rep2((1, HID)),
            rep2((HID, D_EDGE)), rep2((1, D_EDGE)),
            rep2((1, D_EDGE)), rep2((1, D_EDGE)),
            rep2((1, D_EDGE)), rep2((1, 1)),
            rep2((1, D_EDGE)), rep2((1, 1)),
        ],
        out_specs=[
            blk(D_EDGE),
            pl.BlockSpec((D_EDGE, _BE), lambda i: (0, i)),
            pl.BlockSpec((D_EDGE, _BE), lambda i: (0, i)),
        ],
        out_shape=[
            jax.ShapeDtypeStruct((E, D_EDGE), jnp.float32),
            jax.ShapeDtypeStruct((D_EDGE, E), jnp.float32),
            jax.ShapeDtypeStruct((D_EDGE, E), jnp.float32),
        ],
    )(edges, gs, gr, rr, We1e, be1, We2, be2, We3, be3,
      ge, bge, wra, bra, wsa, bsa)


def _scatter_call(wst_f, wrt_f, senders, receivers):
    mesh = plsc.VectorSubcoreMesh(core_axis_name="c", subcore_axis_name="s")
    cp = pltpu.CompilerParams()
    if "needs_layout_passes" in pltpu.CompilerParams.__dataclass_fields__:
        cp = dataclasses.replace(cp, needs_layout_passes=False)

    @functools.partial(
        pl.kernel,
        mesh=mesh,
        compiler_params=cp,
        out_type=[
            jax.ShapeDtypeStruct((D_EDGE * _NP,), jnp.float32),
            jax.ShapeDtypeStruct((D_EDGE * _NP,), jnp.float32),
        ],
        scratch_types=[
            pltpu.VMEM((_SCC,), jnp.int32),
            pltpu.VMEM((_SCC,), jnp.float32),
            pltpu.VMEM((_NP,), jnp.float32),
        ],
    )
    def k(ws_h, wr_h, s_h, r_h, sa_h, ra_h, idx_v, col_v, acc_v):
        cid = lax.axis_index("c")
        sid = lax.axis_index("s")
        zero16 = jnp.zeros((16,), jnp.float32)

        @pl.loop(0, _NP // 16)
        def _(i):
            acc_v[pl.ds(i * 16, 16)] = zero16

        def run(dat_h, i_h, o_h):
            colbase = sid * E

            @pl.loop(0, _SNC)
            def _(c):
                off = c * _SCC
                pltpu.sync_copy(i_h.at[pl.ds(off, _SCC)], idx_v)
                pltpu.sync_copy(dat_h.at[pl.ds(colbase + off, _SCC)], col_v)

                @pl.loop(0, _SNG)
                def _(g):
                    iv = idx_v[pl.ds(g * 16, 16)]
                    vv = col_v[pl.ds(g * 16, 16)]
                    plsc.addupdate_scatter(acc_v, [iv], vv)

            pltpu.sync_copy(acc_v, o_h.at[pl.ds(sid * _NP, _NP)])

        @pl.when(cid == 0)
        def _():
            run(ws_h, s_h, sa_h)

        @pl.when(cid == 1)
        def _():
            run(wr_h, r_h, ra_h)

    return k(wst_f, wrt_f, senders, receivers)


def _node_kernel(n_ref, ra_ref, sa_ref,
                 w1a_ref, w1b_ref, w1c_ref, b1_ref,
                 w2_ref, b2_ref, w3_ref, b3_ref, gn_ref, bgn_ref, out_ref):
    x = n_ref[...]
    h = (_mm(x, w1a_ref[...]) + _mmT(ra_ref[...], w1b_ref[...])
         + _mmT(sa_ref[...], w1c_ref[...]) + b1_ref[...])
    h = h * jax.nn.sigmoid(h)
    h = _mm(h, w2_ref[...]) + b2_ref[...]
    h = h * jax.nn.sigmoid(h)
    h = _mm(h, w3_ref[...]) + b3_ref[...]
    mu = jnp.mean(h, axis=1, keepdims=True)
    d = h - mu
    var = jnp.mean(d * d, axis=1, keepdims=True)
    un = d * jax.lax.rsqrt(var + 1e-5) * gn_ref[...] + bgn_ref[...]
    out_ref[...] = x + un


def _node_call(nodes, recv_agg, sent_agg, Wn1a, Wn1b, Wn1c, bn1,
               Wn2, bn2, Wn3, bn3, gn, bgn):
    rep2 = lambda shape: pl.BlockSpec(shape, lambda i: (0, 0))
    blk = lambda w: pl.BlockSpec((_BN, w), lambda i: (i, 0))
    blkp = lambda w: pl.BlockSpec((_BNP, w), lambda i: (i, 0))
    return pl.pallas_call(
        _node_kernel,
        grid=(_NP // _BNP,),
        in_specs=[
            blkp(D_NODE),
            pl.BlockSpec((D_EDGE, _BNP), lambda i: (0, i)),
            pl.BlockSpec((D_EDGE, _BNP), lambda i: (0, i)),
            rep2((D_NODE, HID)), rep2((D_EDGE, HID)), rep2((D_EDGE, HID)),
            rep2((1, HID)),
            rep2((HID, HID)), rep2((1, HID)),
            rep2((HID, D_NODE)), rep2((1, D_NODE)),
            rep2((1, D_NODE)), rep2((1, D_NODE)),
        ],
        out_specs=blkp(D_NODE),
        out_shape=jax.ShapeDtypeStruct((_NP, D_NODE), jnp.float32),
    )(nodes, recv_agg, sent_agg, Wn1a, Wn1b, Wn1c, bn1,
      Wn2, bn2, Wn3, bn3, gn, bgn)


def kernel(nodes, edges, r, senders, receivers,
           We1, be1, We2, be2, We3, be3, ge, bge,
           Wn1, bn1, Wn2, bn2, Wn3, bn3, gn, bgn,
           Wra, bra, Wsa, bsa):
    We1e = We1[:D_EDGE]
    We1s = We1[D_EDGE:D_EDGE + D_NODE]
    We1r = We1[D_EDGE + D_NODE:]

    ns1, nr1 = _tables_call(nodes, We1s, We1r)
    gs, gr = _gather_call(ns1, nr1, senders, receivers)

    new_edges, wst, wrt = _edge_call(
        edges, gs, gr, r.reshape(E, 1),
        We1e, be1.reshape(1, HID), We2, be2.reshape(1, HID),
        We3, be3.reshape(1, D_EDGE),
        ge.reshape(1, D_EDGE), bge.reshape(1, D_EDGE),
        Wra.reshape(1, D_EDGE), bra.reshape(1, 1),
        Wsa.reshape(1, D_EDGE), bsa.reshape(1, 1))

    sat_f, rat_f = _scatter_call(wst.reshape(D_EDGE * E), wrt.reshape(D_EDGE * E),
                                 senders, receivers)
    sent_agg = sat_f.reshape(D_EDGE, _NP)
    recv_agg = rat_f.reshape(D_EDGE, _NP)

    Wn1a = Wn1[:D_NODE]
    Wn1b = Wn1[D_NODE:D_NODE + D_EDGE]
    Wn1c = Wn1[D_NODE + D_EDGE:]
    nodes_p = jnp.pad(nodes, ((0, _NP - N), (0, 0)))
    new_nodes_p = _node_call(
        nodes_p, recv_agg, sent_agg, Wn1a, Wn1b, Wn1c,
        bn1.reshape(1, HID), Wn2, bn2.reshape(1, HID),
        Wn3, bn3.reshape(1, D_NODE),
        gn.reshape(1, D_NODE), bgn.reshape(1, D_NODE))

    return (new_nodes_p[:N], new_edges)


# speedup vs baseline: 3.1441x; 1.1098x over previous
import dataclasses
import functools

import jax
import jax.numpy as jnp
from jax import lax
from jax.experimental import pallas as pl
from jax.experimental.pallas import tpu as pltpu
from jax.experimental.pallas import tpu_sc as plsc

N = 10000
E = 320000
D_NODE = 128
D_EDGE = 16
HID = 128
R_MAX = 6.0

_NC = 2
_NS = 16
_NW = _NC * _NS

_GCH = 80
_EPW = E // _NW

_SCC = 20000
_SNC = E // _SCC
_SNG = _SCC // 16

_NP = 10240
_BN = 400
_BNP = 512
_BE = 3200


def _mm(a, b):
    return jax.lax.dot_general(a, b, (((1,), (0,)), ((), ())),
                               preferred_element_type=jnp.float32)


def _mmT(a, b):
    return jax.lax.dot_general(a, b, (((0,), (0,)), ((), ())),
                               preferred_element_type=jnp.float32)


def _tables_kernel(nodes_ref, ws_ref, wr_ref, ns_ref, nr_ref):
    x = nodes_ref[...]
    ns_ref[...] = _mm(x, ws_ref[...])
    nr_ref[...] = _mm(x, wr_ref[...])


def _tables_call(nodes, We1s, We1r):
    return pl.pallas_call(
        _tables_kernel,
        grid=(N // _BN,),
        in_specs=[
            pl.BlockSpec((_BN, D_NODE), lambda i: (i, 0)),
            pl.BlockSpec((D_NODE, HID), lambda i: (0, 0)),
            pl.BlockSpec((D_NODE, HID), lambda i: (0, 0)),
        ],
        out_specs=[
            pl.BlockSpec((_BN, HID), lambda i: (i, 0)),
            pl.BlockSpec((_BN, HID), lambda i: (i, 0)),
        ],
        out_shape=[
            jax.ShapeDtypeStruct((N, HID), jnp.float32),
            jax.ShapeDtypeStruct((N, HID), jnp.float32),
        ],
    )(nodes, We1s, We1r)


def _gather_call(ns1, nr1, senders, receivers):
    mesh = plsc.VectorSubcoreMesh(core_axis_name="c", subcore_axis_name="s")

    @functools.partial(
        pl.kernel,
        mesh=mesh,
        out_type=[
            jax.ShapeDtypeStruct((E, HID), jnp.float32),
            jax.ShapeDtypeStruct((E, HID), jnp.float32),
        ],
        scratch_types=[
            pltpu.VMEM((_GCH,), jnp.int32),
            pltpu.VMEM((_GCH,), jnp.int32),
            pltpu.VMEM((_GCH, HID), jnp.float32),
            pltpu.VMEM((_GCH, HID), jnp.float32),
            pltpu.SemaphoreType.DMA,
            pltpu.SemaphoreType.DMA,
        ],
    )
    def k(ns1_h, nr1_h, s_h, r_h, gs_h, gr_h, is_v, ir_v, rs_v, rr_v,
          sem_g, sem_w):
        wid = lax.axis_index("s") * _NC + lax.axis_index("c")
        base = wid * _EPW

        @pl.loop(0, _EPW, step=_GCH)
        def _(off):
            sl = pl.ds(base + off, _GCH)
            pltpu.sync_copy(s_h.at[sl], is_v)
            pltpu.sync_copy(r_h.at[sl], ir_v)
            g1 = pltpu.async_copy(ns1_h.at[is_v], rs_v, sem_g)
            g2 = pltpu.async_copy(nr1_h.at[ir_v], rr_v, sem_g)
            g1.wait()
            g2.wait()
            w1 = pltpu.async_copy(rs_v, gs_h.at[sl], sem_w)
            w2 = pltpu.async_copy(rr_v, gr_h.at[sl], sem_w)
            w1.wait()
            w2.wait()

    return k(ns1, nr1, senders, receivers)


def _cut_kernel(r_ref, out_ref):
    r = r_ref[...]
    x = r * (1.0 / R_MAX)
    x2 = x * x
    x4 = x2 * x2
    x5 = x4 * x
    x6 = x5 * x
    envelope = 1.0 - 15.0 * x4 + 24.0 * x5 - 10.0 * x6
    out_ref[...] = jnp.where(r < R_MAX, envelope, 0.0)


def _cut_call(r2):
    rows = E // 128
    return pl.pallas_call(
        _cut_kernel,
        grid=(1,),
        in_specs=[pl.BlockSpec((rows, 128), lambda i: (0, 0))],
        out_specs=pl.BlockSpec((rows, 128), lambda i: (0, 0)),
        out_shape=jax.ShapeDtypeStruct((rows, 128), jnp.float32),
    )(r2)


def _edge_kernel(e_ref, gs_ref, gr_ref, cut_ref,
                 we1_ref, be1_ref, we2_ref, be2_ref, we3_ref, be3_ref,
                 ge_ref, bge_ref, wra_ref, bra_ref, wsa_ref, bsa_ref,
                 ne_ref, ws_ref, wr_ref):
    e = e_ref[...]
    cut = _mm(cut_ref[...], jnp.ones((1, D_EDGE), dtype=jnp.float32))
    ra = jax.nn.sigmoid(_mm(e, wra_ref[...]) + bra_ref[0, 0]) * cut
    sa = jax.nn.sigmoid(_mm(e, wsa_ref[...]) + bsa_ref[0, 0]) * cut
    h = _mm(e, we1_ref[...]) + gs_ref[...] + gr_ref[...] + be1_ref[...]
    h = h * jax.nn.sigmoid(h)
    h = _mm(h, we2_ref[...]) + be2_ref[...]
    h = h * jax.nn.sigmoid(h)
    h = _mm(h, we3_ref[...]) + be3_ref[...]
    avg16 = jnp.full((D_EDGE, D_EDGE), 1.0 / D_EDGE, dtype=jnp.float32)
    mu = _mm(h, avg16)
    d = h - mu
    var = _mm(d * d, avg16)
    ue = d * jax.lax.rsqrt(var + 1e-5) * ge_ref[...] + bge_ref[...]
    ne_ref[...] = e + ue
    ws_ref[...] = (ue * sa).T
    wr_ref[...] = (ue * ra).T


def _edge_call(edges, gs, gr, cut, We1e, be1, We2, be2, We3, be3,
               ge, bge, wra, bra, wsa, bsa):
    rep2 = lambda shape: pl.BlockSpec(shape, lambda i: (0, 0))
    blk = lambda w: pl.BlockSpec((_BE, w), lambda i: (i, 0))
    return pl.pallas_call(
        _edge_kernel,
        grid=(E // _BE,),
        in_specs=[
            blk(D_EDGE), blk(HID), blk(HID), blk(1),
            rep2((D_EDGE, HID)), rep2((1, HID)),
            rep2((HID, HID)), rep2((1, HID)),
            rep2((HID, D_EDGE)), rep2((1, D_EDGE)),
            rep2((1, D_EDGE)), rep2((1, D_EDGE)),
            rep2((D_EDGE, D_EDGE)), rep2((1, 1)),
            rep2((D_EDGE, D_EDGE)), rep2((1, 1)),
        ],
        out_specs=[
            blk(D_EDGE),
            pl.BlockSpec((D_EDGE, _BE), lambda i: (0, i)),
            pl.BlockSpec((D_EDGE, _BE), lambda i: (0, i)),
        ],
        out_shape=[
            jax.ShapeDtypeStruct((E, D_EDGE), jnp.float32),
            jax.ShapeDtypeStruct((D_EDGE, E), jnp.float32),
            jax.ShapeDtypeStruct((D_EDGE, E), jnp.float32),
        ],
    )(edges, gs, gr, cut, We1e, be1, We2, be2, We3, be3,
      ge, bge, wra, bra, wsa, bsa)


def _scatter_call(wst_f, wrt_f, senders, receivers):
    mesh = plsc.VectorSubcoreMesh(core_axis_name="c", subcore_axis_name="s")
    cp = pltpu.CompilerParams()
    if "needs_layout_passes" in pltpu.CompilerParams.__dataclass_fields__:
        cp = dataclasses.replace(cp, needs_layout_passes=False)

    @functools.partial(
        pl.kernel,
        mesh=mesh,
        compiler_params=cp,
        out_type=[
            jax.ShapeDtypeStruct((D_EDGE * _NP,), jnp.float32),
            jax.ShapeDtypeStruct((D_EDGE * _NP,), jnp.float32),
        ],
        scratch_types=[
            pltpu.VMEM((_SCC,), jnp.int32),
            pltpu.VMEM((_SCC,), jnp.float32),
            pltpu.VMEM((_NP,), jnp.float32),
        ],
    )
    def k(ws_h, wr_h, s_h, r_h, sa_h, ra_h, idx_v, col_v, acc_v):
        cid = lax.axis_index("c")
        sid = lax.axis_index("s")
        zero16 = jnp.zeros((16,), jnp.float32)

        @pl.loop(0, _NP // 16)
        def _(i):
            acc_v[pl.ds(i * 16, 16)] = zero16

        def run(dat_h, i_h, o_h):
            colbase = sid * E

            @pl.loop(0, _SNC)
            def _(c):
                off = c * _SCC
                pltpu.sync_copy(i_h.at[pl.ds(off, _SCC)], idx_v)
                pltpu.sync_copy(dat_h.at[pl.ds(colbase + off, _SCC)], col_v)

                @pl.loop(0, _SNG)
                def _(g):
                    iv = idx_v[pl.ds(g * 16, 16)]
                    vv = col_v[pl.ds(g * 16, 16)]
                    plsc.addupdate_scatter(acc_v, [iv], vv)

            pltpu.sync_copy(acc_v, o_h.at[pl.ds(sid * _NP, _NP)])

        @pl.when(cid == 0)
        def _():
            run(ws_h, s_h, sa_h)

        @pl.when(cid == 1)
        def _():
            run(wr_h, r_h, ra_h)

    return k(wst_f, wrt_f, senders, receivers)


def _node_kernel(n_ref, ra_ref, sa_ref,
                 w1a_ref, w1b_ref, w1c_ref, b1_ref,
                 w2_ref, b2_ref, w3_ref, b3_ref, gn_ref, bgn_ref, out_ref):
    x = n_ref[...]
    h = (_mm(x, w1a_ref[...]) + _mmT(ra_ref[...], w1b_ref[...])
         + _mmT(sa_ref[...], w1c_ref[...]) + b1_ref[...])
    h = h * jax.nn.sigmoid(h)
    h = _mm(h, w2_ref[...]) + b2_ref[...]
    h = h * jax.nn.sigmoid(h)
    h = _mm(h, w3_ref[...]) + b3_ref[...]
    mu = jnp.mean(h, axis=1, keepdims=True)
    d = h - mu
    var = jnp.mean(d * d, axis=1, keepdims=True)
    un = d * jax.lax.rsqrt(var + 1e-5) * gn_ref[...] + bgn_ref[...]
    out_ref[...] = x + un


def _node_call(nodes, recv_agg, sent_agg, Wn1a, Wn1b, Wn1c, bn1,
               Wn2, bn2, Wn3, bn3, gn, bgn):
    rep2 = lambda shape: pl.BlockSpec(shape, lambda i: (0, 0))
    blk = lambda w: pl.BlockSpec((_BN, w), lambda i: (i, 0))
    blkp = lambda w: pl.BlockSpec((_BNP, w), lambda i: (i, 0))
    return pl.pallas_call(
        _node_kernel,
        grid=(_NP // _BNP,),
        in_specs=[
            blkp(D_NODE),
            pl.BlockSpec((D_EDGE, _BNP), lambda i: (0, i)),
            pl.BlockSpec((D_EDGE, _BNP), lambda i: (0, i)),
            rep2((D_NODE, HID)), rep2((D_EDGE, HID)), rep2((D_EDGE, HID)),
            rep2((1, HID)),
            rep2((HID, HID)), rep2((1, HID)),
            rep2((HID, D_NODE)), rep2((1, D_NODE)),
            rep2((1, D_NODE)), rep2((1, D_NODE)),
        ],
        out_specs=blkp(D_NODE),
        out_shape=jax.ShapeDtypeStruct((_NP, D_NODE), jnp.float32),
    )(nodes, recv_agg, sent_agg, Wn1a, Wn1b, Wn1c, bn1,
      Wn2, bn2, Wn3, bn3, gn, bgn)


def kernel(nodes, edges, r, senders, receivers,
           We1, be1, We2, be2, We3, be3, ge, bge,
           Wn1, bn1, Wn2, bn2, Wn3, bn3, gn, bgn,
           Wra, bra, Wsa, bsa):
    We1e = We1[:D_EDGE]
    We1s = We1[D_EDGE:D_EDGE + D_NODE]
    We1r = We1[D_EDGE + D_NODE:]

    ns1, nr1 = _tables_call(nodes, We1s, We1r)
    gs, gr = _gather_call(ns1, nr1, senders, receivers)
    cut = _cut_call(r.reshape(E // 128, 128)).reshape(E, 1)

    new_edges, wst, wrt = _edge_call(
        edges, gs, gr, cut,
        We1e, be1.reshape(1, HID), We2, be2.reshape(1, HID),
        We3, be3.reshape(1, D_EDGE),
        ge.reshape(1, D_EDGE), bge.reshape(1, D_EDGE),
        jnp.tile(Wra, (1, D_EDGE)), bra.reshape(1, 1),
        jnp.tile(Wsa, (1, D_EDGE)), bsa.reshape(1, 1))

    sat_f, rat_f = _scatter_call(wst.reshape(D_EDGE * E), wrt.reshape(D_EDGE * E),
                                 senders, receivers)
    sent_agg = sat_f.reshape(D_EDGE, _NP)
    recv_agg = rat_f.reshape(D_EDGE, _NP)

    Wn1a = Wn1[:D_NODE]
    Wn1b = Wn1[D_NODE:D_NODE + D_EDGE]
    Wn1c = Wn1[D_NODE + D_EDGE:]
    nodes_p = jnp.pad(nodes, ((0, _NP - N), (0, 0)))
    new_nodes_p = _node_call(
        nodes_p, recv_agg, sent_agg, Wn1a, Wn1b, Wn1c,
        bn1.reshape(1, HID), Wn2, bn2.reshape(1, HID),
        Wn3, bn3.reshape(1, D_NODE),
        gn.reshape(1, D_NODE), bgn.reshape(1, D_NODE))

    return (new_nodes_p[:N], new_edges)


# speedup vs baseline: 3.5289x; 1.1224x over previous
import dataclasses
import functools

import jax
import jax.numpy as jnp
from jax import lax
from jax.experimental import pallas as pl
from jax.experimental.pallas import tpu as pltpu
from jax.experimental.pallas import tpu_sc as plsc

N = 10000
E = 320000
D_NODE = 128
D_EDGE = 16
HID = 128
R_MAX = 6.0

_NC = 2
_NS = 16
_NW = _NC * _NS

_GCH = 80
_EPW = E // _NW

_SCC = 20000
_SNC = E // _SCC
_SNG = _SCC // 16

_NP = 10240
_BN = 400
_BNP = 512
_BE = 3200


def _mm(a, b):
    return jax.lax.dot_general(a, b, (((1,), (0,)), ((), ())),
                               preferred_element_type=jnp.float32)


def _mmT(a, b):
    return jax.lax.dot_general(a, b, (((0,), (0,)), ((), ())),
                               preferred_element_type=jnp.float32)


def _tables_kernel(nodes_ref, ws_ref, wr_ref, ns_ref, nr_ref):
    x = nodes_ref[...]
    ns_ref[...] = _mm(x, ws_ref[...])
    nr_ref[...] = _mm(x, wr_ref[...])


def _tables_call(nodes, We1s, We1r):
    return pl.pallas_call(
        _tables_kernel,
        grid=(N // _BN,),
        in_specs=[
            pl.BlockSpec((_BN, D_NODE), lambda i: (i, 0)),
            pl.BlockSpec((D_NODE, HID), lambda i: (0, 0)),
            pl.BlockSpec((D_NODE, HID), lambda i: (0, 0)),
        ],
        out_specs=[
            pl.BlockSpec((_BN, HID), lambda i: (i, 0)),
            pl.BlockSpec((_BN, HID), lambda i: (i, 0)),
        ],
        out_shape=[
            jax.ShapeDtypeStruct((N, HID), jnp.float32),
            jax.ShapeDtypeStruct((N, HID), jnp.float32),
        ],
    )(nodes, We1s, We1r)


def _gather_call(ns1, nr1, senders, receivers):
    mesh = plsc.VectorSubcoreMesh(core_axis_name="c", subcore_axis_name="s")

    @functools.partial(
        pl.kernel,
        mesh=mesh,
        out_type=[
            jax.ShapeDtypeStruct((E, HID), jnp.float32),
            jax.ShapeDtypeStruct((E, HID), jnp.float32),
        ],
        scratch_types=[
            pltpu.VMEM((_EPW,), jnp.int32),
            pltpu.VMEM((_EPW,), jnp.int32),
            pltpu.VMEM((_GCH, HID), jnp.float32),
            pltpu.VMEM((_GCH, HID), jnp.float32),
            pltpu.VMEM((_GCH, HID), jnp.float32),
            pltpu.VMEM((_GCH, HID), jnp.float32),
            pltpu.SemaphoreType.DMA,
            pltpu.SemaphoreType.DMA,
            pltpu.SemaphoreType.DMA,
        ],
    )
    def k(ns1_h, nr1_h, s_h, r_h, gs_h, gr_h, is_v, ir_v,
          rs_a, rr_a, rs_b, rr_b, sem_a, sem_b, sem_w):
        wid = lax.axis_index("s") * _NC + lax.axis_index("c")
        base = wid * _EPW
        pltpu.sync_copy(s_h.at[pl.ds(base, _EPW)], is_v)
        pltpu.sync_copy(r_h.at[pl.ds(base, _EPW)], ir_v)

        def gather(off, rs_v, rr_v, sem):
            g1 = pltpu.async_copy(ns1_h.at[is_v.at[pl.ds(off, _GCH)]],
                                  rs_v, sem)
            g2 = pltpu.async_copy(nr1_h.at[ir_v.at[pl.ds(off, _GCH)]],
                                  rr_v, sem)
            return g1, g2

        def write(off, rs_v, rr_v):
            sl = pl.ds(base + off, _GCH)
            w1 = pltpu.async_copy(rs_v, gs_h.at[sl], sem_w)
            w2 = pltpu.async_copy(rr_v, gr_h.at[sl], sem_w)
            return w1, w2

        @pl.loop(0, (_EPW // _GCH) // 2)
        def _(p):
            off_a = p * (2 * _GCH)
            off_b = off_a + _GCH
            ga1, ga2 = gather(off_a, rs_a, rr_a, sem_a)
            gb1, gb2 = gather(off_b, rs_b, rr_b, sem_b)
            ga1.wait()
            ga2.wait()
            wa1, wa2 = write(off_a, rs_a, rr_a)
            gb1.wait()
            gb2.wait()
            wb1, wb2 = write(off_b, rs_b, rr_b)
            wa1.wait()
            wa2.wait()
            wb1.wait()
            wb2.wait()

        tail = ((_EPW // _GCH) // 2) * 2 * _GCH
        g1, g2 = gather(tail, rs_a, rr_a, sem_a)
        g1.wait()
        g2.wait()
        w1, w2 = write(tail, rs_a, rr_a)
        w1.wait()
        w2.wait()

    return k(ns1, nr1, senders, receivers)


def _cut_kernel(r_ref, out_ref):
    r = r_ref[...]
    x = r * (1.0 / R_MAX)
    x2 = x * x
    x4 = x2 * x2
    x5 = x4 * x
    x6 = x5 * x
    envelope = 1.0 - 15.0 * x4 + 24.0 * x5 - 10.0 * x6
    out_ref[...] = jnp.where(r < R_MAX, envelope, 0.0)


def _cut_call(r2):
    rows = E // 128
    return pl.pallas_call(
        _cut_kernel,
        grid=(1,),
        in_specs=[pl.BlockSpec((rows, 128), lambda i: (0, 0))],
        out_specs=pl.BlockSpec((rows, 128), lambda i: (0, 0)),
        out_shape=jax.ShapeDtypeStruct((rows, 128), jnp.float32),
    )(r2)


def _edge_kernel(e_ref, gs_ref, gr_ref, cut_ref,
                 we1_ref, be1_ref, we2_ref, be2_ref, we3_ref, be3_ref,
                 ge_ref, bge_ref, wra_ref, bra_ref, wsa_ref, bsa_ref,
                 ne_ref, ws_ref, wr_ref):
    e = e_ref[...]
    cut = _mm(cut_ref[...], jnp.ones((1, D_EDGE), dtype=jnp.float32))
    ra = jax.nn.sigmoid(_mm(e, wra_ref[...]) + bra_ref[0, 0]) * cut
    sa = jax.nn.sigmoid(_mm(e, wsa_ref[...]) + bsa_ref[0, 0]) * cut
    h = _mm(e, we1_ref[...]) + gs_ref[...] + gr_ref[...] + be1_ref[...]
    h = h * jax.nn.sigmoid(h)
    h = _mm(h, we2_ref[...]) + be2_ref[...]
    h = h * jax.nn.sigmoid(h)
    h = _mm(h, we3_ref[...]) + be3_ref[...]
    avg16 = jnp.full((D_EDGE, D_EDGE), 1.0 / D_EDGE, dtype=jnp.float32)
    mu = _mm(h, avg16)
    d = h - mu
    var = _mm(d * d, avg16)
    ue = d * jax.lax.rsqrt(var + 1e-5) * ge_ref[...] + bge_ref[...]
    ne_ref[...] = e + ue
    ws_ref[...] = (ue * sa).T
    wr_ref[...] = (ue * ra).T


def _edge_call(edges, gs, gr, cut, We1e, be1, We2, be2, We3, be3,
               ge, bge, wra, bra, wsa, bsa):
    rep2 = lambda shape: pl.BlockSpec(shape, lambda i: (0, 0))
    blk = lambda w: pl.BlockSpec((_BE, w), lambda i: (i, 0))
    return pl.pallas_call(
        _edge_kernel,
        grid=(E // _BE,),
        in_specs=[
            blk(D_EDGE), blk(HID), blk(HID), blk(1),
            rep2((D_EDGE, HID)), rep2((1, HID)),
            rep2((HID, HID)), rep2((1, HID)),
            rep2((HID, D_EDGE)), rep2((1, D_EDGE)),
            rep2((1, D_EDGE)), rep2((1, D_EDGE)),
            rep2((D_EDGE, D_EDGE)), rep2((1, 1)),
            rep2((D_EDGE, D_EDGE)), rep2((1, 1)),
        ],
        out_specs=[
            blk(D_EDGE),
            pl.BlockSpec((D_EDGE, _BE), lambda i: (0, i)),
            pl.BlockSpec((D_EDGE, _BE), lambda i: (0, i)),
        ],
        out_shape=[
            jax.ShapeDtypeStruct((E, D_EDGE), jnp.float32),
            jax.ShapeDtypeStruct((D_EDGE, E), jnp.float32),
            jax.ShapeDtypeStruct((D_EDGE, E), jnp.float32),
        ],
    )(edges, gs, gr, cut, We1e, be1, We2, be2, We3, be3,
      ge, bge, wra, bra, wsa, bsa)


def _scatter_call(wst_f, wrt_f, senders, receivers):
    mesh = plsc.VectorSubcoreMesh(core_axis_name="c", subcore_axis_name="s")
    cp = pltpu.CompilerParams()
    if "needs_layout_passes" in pltpu.CompilerParams.__dataclass_fields__:
        cp = dataclasses.replace(cp, needs_layout_passes=False)

    @functools.partial(
        pl.kernel,
        mesh=mesh,
        compiler_params=cp,
        out_type=[
            jax.ShapeDtypeStruct((D_EDGE * _NP,), jnp.float32),
            jax.ShapeDtypeStruct((D_EDGE * _NP,), jnp.float32),
        ],
        scratch_types=[
            pltpu.VMEM((_SCC,), jnp.int32),
            pltpu.VMEM((_SCC,), jnp.float32),
            pltpu.VMEM((_NP,), jnp.float32),
        ],
    )
    def k(ws_h, wr_h, s_h, r_h, sa_h, ra_h, idx_v, col_v, acc_v):
        cid = lax.axis_index("c")
        sid = lax.axis_index("s")
        zero16 = jnp.zeros((16,), jnp.float32)

        @pl.loop(0, _NP // 16)
        def _(i):
            acc_v[pl.ds(i * 16, 16)] = zero16

        def run(dat_h, i_h, o_h):
            colbase = sid * E

            @pl.loop(0, _SNC)
            def _(c):
                off = c * _SCC
                pltpu.sync_copy(i_h.at[pl.ds(off, _SCC)], idx_v)
                pltpu.sync_copy(dat_h.at[pl.ds(colbase + off, _SCC)], col_v)

                @pl.loop(0, _SNG)
                def _(g):
                    iv = idx_v[pl.ds(g * 16, 16)]
                    vv = col_v[pl.ds(g * 16, 16)]
                    plsc.addupdate_scatter(acc_v, [iv], vv)

            pltpu.sync_copy(acc_v, o_h.at[pl.ds(sid * _NP, _NP)])

        @pl.when(cid == 0)
        def _():
            run(ws_h, s_h, sa_h)

        @pl.when(cid == 1)
        def _():
            run(wr_h, r_h, ra_h)

    return k(wst_f, wrt_f, senders, receivers)


def _node_kernel(n_ref, ra_ref, sa_ref,
                 w1a_ref, w1b_ref, w1c_ref, b1_ref,
                 w2_ref, b2_ref, w3_ref, b3_ref, gn_ref, bgn_ref, out_ref):
    x = n_ref[...]
    h = (_mm(x, w1a_ref[...]) + _mmT(ra_ref[...], w1b_ref[...])
         + _mmT(sa_ref[...], w1c_ref[...]) + b1_ref[...])
    h = h * jax.nn.sigmoid(h)
    h = _mm(h, w2_ref[...]) + b2_ref[...]
    h = h * jax.nn.sigmoid(h)
    h = _mm(h, w3_ref[...]) + b3_ref[...]
    mu = jnp.mean(h, axis=1, keepdims=True)
    d = h - mu
    var = jnp.mean(d * d, axis=1, keepdims=True)
    un = d * jax.lax.rsqrt(var + 1e-5) * gn_ref[...] + bgn_ref[...]
    out_ref[...] = x + un


def _node_call(nodes, recv_agg, sent_agg, Wn1a, Wn1b, Wn1c, bn1,
               Wn2, bn2, Wn3, bn3, gn, bgn):
    rep2 = lambda shape: pl.BlockSpec(shape, lambda i: (0, 0))
    blk = lambda w: pl.BlockSpec((_BN, w), lambda i: (i, 0))
    blkp = lambda w: pl.BlockSpec((_BNP, w), lambda i: (i, 0))
    return pl.pallas_call(
        _node_kernel,
        grid=(_NP // _BNP,),
        in_specs=[
            blkp(D_NODE),
            pl.BlockSpec((D_EDGE, _BNP), lambda i: (0, i)),
            pl.BlockSpec((D_EDGE, _BNP), lambda i: (0, i)),
            rep2((D_NODE, HID)), rep2((D_EDGE, HID)), rep2((D_EDGE, HID)),
            rep2((1, HID)),
            rep2((HID, HID)), rep2((1, HID)),
            rep2((HID, D_NODE)), rep2((1, D_NODE)),
            rep2((1, D_NODE)), rep2((1, D_NODE)),
        ],
        out_specs=blkp(D_NODE),
        out_shape=jax.ShapeDtypeStruct((_NP, D_NODE), jnp.float32),
    )(nodes, recv_agg, sent_agg, Wn1a, Wn1b, Wn1c, bn1,
      Wn2, bn2, Wn3, bn3, gn, bgn)


def kernel(nodes, edges, r, senders, receivers,
           We1, be1, We2, be2, We3, be3, ge, bge,
           Wn1, bn1, Wn2, bn2, Wn3, bn3, gn, bgn,
           Wra, bra, Wsa, bsa):
    We1e = We1[:D_EDGE]
    We1s = We1[D_EDGE:D_EDGE + D_NODE]
    We1r = We1[D_EDGE + D_NODE:]

    ns1, nr1 = _tables_call(nodes, We1s, We1r)
    gs, gr = _gather_call(ns1, nr1, senders, receivers)
    cut = _cut_call(r.reshape(E // 128, 128)).reshape(E, 1)

    new_edges, wst, wrt = _edge_call(
        edges, gs, gr, cut,
        We1e, be1.reshape(1, HID), We2, be2.reshape(1, HID),
        We3, be3.reshape(1, D_EDGE),
        ge.reshape(1, D_EDGE), bge.reshape(1, D_EDGE),
        jnp.tile(Wra, (1, D_EDGE)), bra.reshape(1, 1),
        jnp.tile(Wsa, (1, D_EDGE)), bsa.reshape(1, 1))

    sat_f, rat_f = _scatter_call(wst.reshape(D_EDGE * E), wrt.reshape(D_EDGE * E),
                                 senders, receivers)
    sent_agg = sat_f.reshape(D_EDGE, _NP)
    recv_agg = rat_f.reshape(D_EDGE, _NP)

    Wn1a = Wn1[:D_NODE]
    Wn1b = Wn1[D_NODE:D_NODE + D_EDGE]
    Wn1c = Wn1[D_NODE + D_EDGE:]
    nodes_p = jnp.pad(nodes, ((0, _NP - N), (0, 0)))
    new_nodes_p = _node_call(
        nodes_p, recv_agg, sent_agg, Wn1a, Wn1b, Wn1c,
        bn1.reshape(1, HID), Wn2, bn2.reshape(1, HID),
        Wn3, bn3.reshape(1, D_NODE),
        gn.reshape(1, D_NODE), bgn.reshape(1, D_NODE))

    return (new_nodes_p[:N], new_edges)


# speedup vs baseline: 3.5512x; 1.0063x over previous
import dataclasses
import functools

import jax
import jax.numpy as jnp
from jax import lax
from jax.experimental import pallas as pl
from jax.experimental.pallas import tpu as pltpu
from jax.experimental.pallas import tpu_sc as plsc

N = 10000
E = 320000
D_NODE = 128
D_EDGE = 16
HID = 128
R_MAX = 6.0

_NC = 2
_NS = 16
_NW = _NC * _NS

_GCH = 80
_EPW = E // _NW
_NBUF = 4

_SCC = 20000
_SNC = E // _SCC
_SNG = _SCC // 16

_NP = 10240
_BN = 400
_BNP = 512
_BE = 3200


def _mm(a, b):
    return jax.lax.dot_general(a, b, (((1,), (0,)), ((), ())),
                               preferred_element_type=jnp.float32)


def _mmT(a, b):
    return jax.lax.dot_general(a, b, (((0,), (0,)), ((), ())),
                               preferred_element_type=jnp.float32)


def _tables_kernel(nodes_ref, ws_ref, wr_ref, ns_ref, nr_ref):
    x = nodes_ref[...]
    ns_ref[...] = _mm(x, ws_ref[...])
    nr_ref[...] = _mm(x, wr_ref[...])


def _tables_call(nodes, We1s, We1r):
    return pl.pallas_call(
        _tables_kernel,
        grid=(N // _BN,),
        in_specs=[
            pl.BlockSpec((_BN, D_NODE), lambda i: (i, 0)),
            pl.BlockSpec((D_NODE, HID), lambda i: (0, 0)),
            pl.BlockSpec((D_NODE, HID), lambda i: (0, 0)),
        ],
        out_specs=[
            pl.BlockSpec((_BN, HID), lambda i: (i, 0)),
            pl.BlockSpec((_BN, HID), lambda i: (i, 0)),
        ],
        out_shape=[
            jax.ShapeDtypeStruct((N, HID), jnp.float32),
            jax.ShapeDtypeStruct((N, HID), jnp.float32),
        ],
    )(nodes, We1s, We1r)


def _gather_call(ns1, nr1, senders, receivers):
    mesh = plsc.VectorSubcoreMesh(core_axis_name="c", subcore_axis_name="s")

    @functools.partial(
        pl.kernel,
        mesh=mesh,
        out_type=[
            jax.ShapeDtypeStruct((E, HID), jnp.float32),
            jax.ShapeDtypeStruct((E, HID), jnp.float32),
        ],
        scratch_types=(
            [pltpu.VMEM((_EPW,), jnp.int32)] * 2
            + [pltpu.VMEM((_GCH, HID), jnp.float32)] * (2 * _NBUF)
            + [pltpu.SemaphoreType.DMA] * (_NBUF + 1)
        ),
    )
    def k(ns1_h, nr1_h, s_h, r_h, gs_h, gr_h, is_v, ir_v, *rest):
        bufs = rest[:2 * _NBUF]
        sems = rest[2 * _NBUF:]
        sem_w = sems[_NBUF]
        wid = lax.axis_index("s") * _NC + lax.axis_index("c")
        base = wid * _EPW
        pltpu.sync_copy(s_h.at[pl.ds(base, _EPW)], is_v)
        pltpu.sync_copy(r_h.at[pl.ds(base, _EPW)], ir_v)

        def gather(off, b):
            g1 = pltpu.async_copy(ns1_h.at[is_v.at[pl.ds(off, _GCH)]],
                                  bufs[2 * b], sems[b])
            g2 = pltpu.async_copy(nr1_h.at[ir_v.at[pl.ds(off, _GCH)]],
                                  bufs[2 * b + 1], sems[b])
            return g1, g2

        def write(off, b):
            sl = pl.ds(base + off, _GCH)
            w1 = pltpu.async_copy(bufs[2 * b], gs_h.at[sl], sem_w)
            w2 = pltpu.async_copy(bufs[2 * b + 1], gr_h.at[sl], sem_w)
            return w1, w2

        @pl.loop(0, (_EPW // _GCH) // _NBUF)
        def _(q):
            base_off = q * (_NBUF * _GCH)
            gs_pend = [gather(base_off + b * _GCH, b) for b in range(_NBUF)]
            ws_pend = []
            for b in range(_NBUF):
                g1, g2 = gs_pend[b]
                g1.wait()
                g2.wait()
                ws_pend.append(write(base_off + b * _GCH, b))
            for w1, w2 in ws_pend:
                w1.wait()
                w2.wait()

        tail = ((_EPW // _GCH) // _NBUF) * _NBUF * _GCH
        g1, g2 = gather(tail, 0)
        g1.wait()
        g2.wait()
        w1, w2 = write(tail, 0)
        w1.wait()
        w2.wait()

    return k(ns1, nr1, senders, receivers)


def _cut_kernel(r_ref, out_ref):
    r = r_ref[...]
    x = r * (1.0 / R_MAX)
    x2 = x * x
    x4 = x2 * x2
    x5 = x4 * x
    x6 = x5 * x
    envelope = 1.0 - 15.0 * x4 + 24.0 * x5 - 10.0 * x6
    out_ref[...] = jnp.where(r < R_MAX, envelope, 0.0)


def _cut_call(r2):
    rows = E // 128
    return pl.pallas_call(
        _cut_kernel,
        grid=(1,),
        in_specs=[pl.BlockSpec((rows, 128), lambda i: (0, 0))],
        out_specs=pl.BlockSpec((rows, 128), lambda i: (0, 0)),
        out_shape=jax.ShapeDtypeStruct((rows, 128), jnp.float32),
    )(r2)


def _edge_kernel(e_ref, gs_ref, gr_ref, cut_ref,
                 we1_ref, be1_ref, we2_ref, be2_ref, we3_ref, be3_ref,
                 ge_ref, bge_ref, wra_ref, bra_ref, wsa_ref, bsa_ref,
                 ne_ref, ws_ref, wr_ref):
    e = e_ref[...]
    cut = _mm(cut_ref[...], jnp.ones((1, D_EDGE), dtype=jnp.float32))
    ra = jax.nn.sigmoid(_mm(e, wra_ref[...]) + bra_ref[0, 0]) * cut
    sa = jax.nn.sigmoid(_mm(e, wsa_ref[...]) + bsa_ref[0, 0]) * cut
    h = _mm(e, we1_ref[...]) + gs_ref[...] + gr_ref[...] + be1_ref[...]
    h = h * jax.nn.sigmoid(h)
    h = _mm(h, we2_ref[...]) + be2_ref[...]
    h = h * jax.nn.sigmoid(h)
    h = _mm(h, we3_ref[...]) + be3_ref[...]
    avg16 = jnp.full((D_EDGE, D_EDGE), 1.0 / D_EDGE, dtype=jnp.float32)
    mu = _mm(h, avg16)
    d = h - mu
    var = _mm(d * d, avg16)
    ue = d * jax.lax.rsqrt(var + 1e-5) * ge_ref[...] + bge_ref[...]
    ne_ref[...] = e + ue
    ws_ref[...] = (ue * sa).T
    wr_ref[...] = (ue * ra).T


def _edge_call(edges, gs, gr, cut, We1e, be1, We2, be2, We3, be3,
               ge, bge, wra, bra, wsa, bsa):
    rep2 = lambda shape: pl.BlockSpec(shape, lambda i: (0, 0))
    blk = lambda w: pl.BlockSpec((_BE, w), lambda i: (i, 0))
    return pl.pallas_call(
        _edge_kernel,
        grid=(E // _BE,),
        in_specs=[
            blk(D_EDGE), blk(HID), blk(HID), blk(1),
            rep2((D_EDGE, HID)), rep2((1, HID)),
            rep2((HID, HID)), rep2((1, HID)),
            rep2((HID, D_EDGE)), rep2((1, D_EDGE)),
            rep2((1, D_EDGE)), rep2((1, D_EDGE)),
            rep2((D_EDGE, D_EDGE)), rep2((1, 1)),
            rep2((D_EDGE, D_EDGE)), rep2((1, 1)),
        ],
        out_specs=[
            blk(D_EDGE),
            pl.BlockSpec((D_EDGE, _BE), lambda i: (0, i)),
            pl.BlockSpec((D_EDGE, _BE), lambda i: (0, i)),
        ],
        out_shape=[
            jax.ShapeDtypeStruct((E, D_EDGE), jnp.float32),
            jax.ShapeDtypeStruct((D_EDGE, E), jnp.float32),
            jax.ShapeDtypeStruct((D_EDGE, E), jnp.float32),
        ],
    )(edges, gs, gr, cut, We1e, be1, We2, be2, We3, be3,
      ge, bge, wra, bra, wsa, bsa)


def _scatter_call(wst_f, wrt_f, senders, receivers):
    mesh = plsc.VectorSubcoreMesh(core_axis_name="c", subcore_axis_name="s")
    cp = pltpu.CompilerParams()
    if "needs_layout_passes" in pltpu.CompilerParams.__dataclass_fields__:
        cp = dataclasses.replace(cp, needs_layout_passes=False)

    @functools.partial(
        pl.kernel,
        mesh=mesh,
        compiler_params=cp,
        out_type=[
            jax.ShapeDtypeStruct((D_EDGE * _NP,), jnp.float32),
            jax.ShapeDtypeStruct((D_EDGE * _NP,), jnp.float32),
        ],
        scratch_types=[
            pltpu.VMEM((_SCC,), jnp.int32),
            pltpu.VMEM((_SCC,), jnp.float32),
            pltpu.VMEM((_NP,), jnp.float32),
        ],
    )
    def k(ws_h, wr_h, s_h, r_h, sa_h, ra_h, idx_v, col_v, acc_v):
        cid = lax.axis_index("c")
        sid = lax.axis_index("s")
        zero16 = jnp.zeros((16,), jnp.float32)

        @pl.loop(0, _NP // 16)
        def _(i):
            acc_v[pl.ds(i * 16, 16)] = zero16

        def run(dat_h, i_h, o_h):
            colbase = sid * E

            @pl.loop(0, _SNC)
            def _(c):
                off = c * _SCC
                pltpu.sync_copy(i_h.at[pl.ds(off, _SCC)], idx_v)
                pltpu.sync_copy(dat_h.at[pl.ds(colbase + off, _SCC)], col_v)

                @pl.loop(0, _SNG // 5)
                def _(g5):
                    for u in range(5):
                        g = g5 * 5 + u
                        iv = idx_v[pl.ds(g * 16, 16)]
                        vv = col_v[pl.ds(g * 16, 16)]
                        plsc.addupdate_scatter(acc_v, [iv], vv)

            pltpu.sync_copy(acc_v, o_h.at[pl.ds(sid * _NP, _NP)])

        @pl.when(cid == 0)
        def _():
            run(ws_h, s_h, sa_h)

        @pl.when(cid == 1)
        def _():
            run(wr_h, r_h, ra_h)

    return k(wst_f, wrt_f, senders, receivers)


def _node_kernel(n_ref, ra_ref, sa_ref,
                 w1a_ref, w1b_ref, w1c_ref, b1_ref,
                 w2_ref, b2_ref, w3_ref, b3_ref, gn_ref, bgn_ref, out_ref):
    x = n_ref[...]
    h = (_mm(x, w1a_ref[...]) + _mmT(ra_ref[...], w1b_ref[...])
         + _mmT(sa_ref[...], w1c_ref[...]) + b1_ref[...])
    h = h * jax.nn.sigmoid(h)
    h = _mm(h, w2_ref[...]) + b2_ref[...]
    h = h * jax.nn.sigmoid(h)
    h = _mm(h, w3_ref[...]) + b3_ref[...]
    mu = jnp.mean(h, axis=1, keepdims=True)
    d = h - mu
    var = jnp.mean(d * d, axis=1, keepdims=True)
    un = d * jax.lax.rsqrt(var + 1e-5) * gn_ref[...] + bgn_ref[...]
    out_ref[...] = x + un


def _node_call(nodes, recv_agg, sent_agg, Wn1a, Wn1b, Wn1c, bn1,
               Wn2, bn2, Wn3, bn3, gn, bgn):
    rep2 = lambda shape: pl.BlockSpec(shape, lambda i: (0, 0))
    blk = lambda w: pl.BlockSpec((_BN, w), lambda i: (i, 0))
    blkp = lambda w: pl.BlockSpec((_BNP, w), lambda i: (i, 0))
    return pl.pallas_call(
        _node_kernel,
        grid=(_NP // _BNP,),
        in_specs=[
            blkp(D_NODE),
            pl.BlockSpec((D_EDGE, _BNP), lambda i: (0, i)),
            pl.BlockSpec((D_EDGE, _BNP), lambda i: (0, i)),
            rep2((D_NODE, HID)), rep2((D_EDGE, HID)), rep2((D_EDGE, HID)),
            rep2((1, HID)),
            rep2((HID, HID)), rep2((1, HID)),
            rep2((HID, D_NODE)), rep2((1, D_NODE)),
            rep2((1, D_NODE)), rep2((1, D_NODE)),
        ],
        out_specs=blkp(D_NODE),
        out_shape=jax.ShapeDtypeStruct((_NP, D_NODE), jnp.float32),
    )(nodes, recv_agg, sent_agg, Wn1a, Wn1b, Wn1c, bn1,
      Wn2, bn2, Wn3, bn3, gn, bgn)


def kernel(nodes, edges, r, senders, receivers,
           We1, be1, We2, be2, We3, be3, ge, bge,
           Wn1, bn1, Wn2, bn2, Wn3, bn3, gn, bgn,
           Wra, bra, Wsa, bsa):
    We1e = We1[:D_EDGE]
    We1s = We1[D_EDGE:D_EDGE + D_NODE]
    We1r = We1[D_EDGE + D_NODE:]

    ns1, nr1 = _tables_call(nodes, We1s, We1r)
    gs, gr = _gather_call(ns1, nr1, senders, receivers)
    cut = _cut_call(r.reshape(E // 128, 128)).reshape(E, 1)

    new_edges, wst, wrt = _edge_call(
        edges, gs, gr, cut,
        We1e, be1.reshape(1, HID), We2, be2.reshape(1, HID),
        We3, be3.reshape(1, D_EDGE),
        ge.reshape(1, D_EDGE), bge.reshape(1, D_EDGE),
        jnp.tile(Wra, (1, D_EDGE)), bra.reshape(1, 1),
        jnp.tile(Wsa, (1, D_EDGE)), bsa.reshape(1, 1))

    sat_f, rat_f = _scatter_call(wst.reshape(D_EDGE * E), wrt.reshape(D_EDGE * E),
                                 senders, receivers)
    sent_agg = sat_f.reshape(D_EDGE, _NP)
    recv_agg = rat_f.reshape(D_EDGE, _NP)

    Wn1a = Wn1[:D_NODE]
    Wn1b = Wn1[D_NODE:D_NODE + D_EDGE]
    Wn1c = Wn1[D_NODE + D_EDGE:]
    nodes_p = jnp.pad(nodes, ((0, _NP - N), (0, 0)))
    new_nodes_p = _node_call(
        nodes_p, recv_agg, sent_agg, Wn1a, Wn1b, Wn1c,
        bn1.reshape(1, HID), Wn2, bn2.reshape(1, HID),
        Wn3, bn3.reshape(1, D_NODE),
        gn.reshape(1, D_NODE), bgn.reshape(1, D_NODE))

    return (new_nodes_p[:N], new_edges)


# speedup vs baseline: 3.7763x; 1.0634x over previous
import dataclasses
import functools

import jax
import jax.numpy as jnp
from jax import lax
from jax.experimental import pallas as pl
from jax.experimental.pallas import tpu as pltpu
from jax.experimental.pallas import tpu_sc as plsc

N = 10000
E = 320000
D_NODE = 128
D_EDGE = 16
HID = 128
R_MAX = 6.0

_NC = 2
_NS = 16
_NW = _NC * _NS

_GCH = 80
_EPW = E // _NW
_NBUF = 4

_SCC = 20000
_SNC = E // _SCC
_SNG = _SCC // 16

_NP = 10240
_BN = 400
_BNP = 512
_BE = 3200


def _mm(a, b):
    return jax.lax.dot_general(a, b, (((1,), (0,)), ((), ())),
                               preferred_element_type=jnp.float32)


def _mmT(a, b):
    return jax.lax.dot_general(a, b, (((0,), (0,)), ((), ())),
                               preferred_element_type=jnp.float32)


def _tables_kernel(nodes_ref, ws_ref, wr_ref, ns_ref, nr_ref):
    x = nodes_ref[...]
    ns_ref[...] = _mm(x, ws_ref[...])
    nr_ref[...] = _mm(x, wr_ref[...])


def _tables_call(nodes, We1s, We1r):
    return pl.pallas_call(
        _tables_kernel,
        grid=(N // _BN,),
        in_specs=[
            pl.BlockSpec((_BN, D_NODE), lambda i: (i, 0)),
            pl.BlockSpec((D_NODE, HID), lambda i: (0, 0)),
            pl.BlockSpec((D_NODE, HID), lambda i: (0, 0)),
        ],
        out_specs=[
            pl.BlockSpec((_BN, HID), lambda i: (i, 0)),
            pl.BlockSpec((_BN, HID), lambda i: (i, 0)),
        ],
        out_shape=[
            jax.ShapeDtypeStruct((N, HID), jnp.float32),
            jax.ShapeDtypeStruct((N, HID), jnp.float32),
        ],
    )(nodes, We1s, We1r)


def _gather_call(ns1, nr1, senders, receivers):
    mesh = plsc.VectorSubcoreMesh(core_axis_name="c", subcore_axis_name="s")

    @functools.partial(
        pl.kernel,
        mesh=mesh,
        out_type=[
            jax.ShapeDtypeStruct((E, HID), jnp.float32),
            jax.ShapeDtypeStruct((E, HID), jnp.float32),
        ],
        scratch_types=(
            [pltpu.VMEM((_EPW,), jnp.int32)] * 2
            + [pltpu.VMEM((_GCH, HID), jnp.float32)] * (2 * _NBUF)
            + [pltpu.SemaphoreType.DMA] * (_NBUF + 1)
        ),
    )
    def k(ns1_h, nr1_h, s_h, r_h, gs_h, gr_h, is_v, ir_v, *rest):
        bufs = rest[:2 * _NBUF]
        sems = rest[2 * _NBUF:]
        sem_w = sems[_NBUF]
        wid = lax.axis_index("s") * _NC + lax.axis_index("c")
        base = wid * _EPW
        pltpu.sync_copy(s_h.at[pl.ds(base, _EPW)], is_v)
        pltpu.sync_copy(r_h.at[pl.ds(base, _EPW)], ir_v)

        def gather(off, b):
            g1 = pltpu.async_copy(ns1_h.at[is_v.at[pl.ds(off, _GCH)]],
                                  bufs[2 * b], sems[b])
            g2 = pltpu.async_copy(nr1_h.at[ir_v.at[pl.ds(off, _GCH)]],
                                  bufs[2 * b + 1], sems[b])
            return g1, g2

        def write(off, b):
            sl = pl.ds(base + off, _GCH)
            w1 = pltpu.async_copy(bufs[2 * b], gs_h.at[sl], sem_w)
            w2 = pltpu.async_copy(bufs[2 * b + 1], gr_h.at[sl], sem_w)
            return w1, w2

        @pl.loop(0, (_EPW // _GCH) // _NBUF)
        def _(q):
            base_off = q * (_NBUF * _GCH)
            gs_pend = [gather(base_off + b * _GCH, b) for b in range(_NBUF)]
            ws_pend = []
            for b in range(_NBUF):
                g1, g2 = gs_pend[b]
                g1.wait()
                g2.wait()
                ws_pend.append(write(base_off + b * _GCH, b))
            for w1, w2 in ws_pend:
                w1.wait()
                w2.wait()

        tail = ((_EPW // _GCH) // _NBUF) * _NBUF * _GCH
        g1, g2 = gather(tail, 0)
        g1.wait()
        g2.wait()
        w1, w2 = write(tail, 0)
        w1.wait()
        w2.wait()

    return k(ns1, nr1, senders, receivers)


def _cut_kernel(r_ref, out_ref):
    r = r_ref[...]
    x = r * (1.0 / R_MAX)
    x2 = x * x
    x4 = x2 * x2
    x5 = x4 * x
    x6 = x5 * x
    envelope = 1.0 - 15.0 * x4 + 24.0 * x5 - 10.0 * x6
    out_ref[...] = jnp.where(r < R_MAX, envelope, 0.0)


def _cut_call(r2):
    rows = E // 128
    return pl.pallas_call(
        _cut_kernel,
        grid=(1,),
        in_specs=[pl.BlockSpec((rows, 128), lambda i: (0, 0))],
        out_specs=pl.BlockSpec((rows, 128), lambda i: (0, 0)),
        out_shape=jax.ShapeDtypeStruct((rows, 128), jnp.float32),
    )(r2)


def _edge_kernel(e_ref, gs_ref, gr_ref, cut_ref,
                 we1_ref, be1_ref, we2_ref, be2_ref, we3_ref, be3_ref,
                 ge_ref, bge_ref, wra_ref, bra_ref, wsa_ref, bsa_ref,
                 ne_ref, ws_ref, wr_ref):
    e = e_ref[...]
    cut = _mm(cut_ref[...], jnp.ones((1, D_EDGE), dtype=jnp.float32))
    ra = jax.nn.sigmoid(_mm(e, wra_ref[...]) + bra_ref[0, 0]) * cut
    sa = jax.nn.sigmoid(_mm(e, wsa_ref[...]) + bsa_ref[0, 0]) * cut
    h = _mm(e, we1_ref[...]) + gs_ref[...] + gr_ref[...] + be1_ref[...]
    h = h * jax.nn.sigmoid(h)
    h = _mm(h, we2_ref[...]) + be2_ref[...]
    h = h * jax.nn.sigmoid(h)
    h = _mm(h, we3_ref[...]) + be3_ref[...]
    avg16 = jnp.full((D_EDGE, D_EDGE), 1.0 / D_EDGE, dtype=jnp.float32)
    mu = _mm(h, avg16)
    d = h - mu
    var = _mm(d * d, avg16)
    ue = d * jax.lax.rsqrt(var + 1e-5) * ge_ref[...] + bge_ref[...]
    ne_ref[...] = e + ue
    ws_ref[...] = (ue * sa).T
    wr_ref[...] = (ue * ra).T


def _edge_call(edges, gs, gr, cut, We1e, be1, We2, be2, We3, be3,
               ge, bge, wra, bra, wsa, bsa):
    rep2 = lambda shape: pl.BlockSpec(shape, lambda i: (0, 0))
    blk = lambda w: pl.BlockSpec((_BE, w), lambda i: (i, 0))
    return pl.pallas_call(
        _edge_kernel,
        grid=(E // _BE,),
        in_specs=[
            blk(D_EDGE), blk(HID), blk(HID), blk(1),
            rep2((D_EDGE, HID)), rep2((1, HID)),
            rep2((HID, HID)), rep2((1, HID)),
            rep2((HID, D_EDGE)), rep2((1, D_EDGE)),
            rep2((1, D_EDGE)), rep2((1, D_EDGE)),
            rep2((D_EDGE, D_EDGE)), rep2((1, 1)),
            rep2((D_EDGE, D_EDGE)), rep2((1, 1)),
        ],
        out_specs=[
            blk(D_EDGE),
            pl.BlockSpec((D_EDGE, _BE), lambda i: (0, i)),
            pl.BlockSpec((D_EDGE, _BE), lambda i: (0, i)),
        ],
        out_shape=[
            jax.ShapeDtypeStruct((E, D_EDGE), jnp.float32),
            jax.ShapeDtypeStruct((D_EDGE, E), jnp.float32),
            jax.ShapeDtypeStruct((D_EDGE, E), jnp.float32),
        ],
    )(edges, gs, gr, cut, We1e, be1, We2, be2, We3, be3,
      ge, bge, wra, bra, wsa, bsa)


def _scatter_call(wst_f, wrt_f, senders, receivers):
    mesh = plsc.VectorSubcoreMesh(core_axis_name="c", subcore_axis_name="s")
    cp = pltpu.CompilerParams()
    if "needs_layout_passes" in pltpu.CompilerParams.__dataclass_fields__:
        cp = dataclasses.replace(cp, needs_layout_passes=False)

    @functools.partial(
        pl.kernel,
        mesh=mesh,
        compiler_params=cp,
        out_type=[
            jax.ShapeDtypeStruct((D_EDGE * _NP,), jnp.float32),
            jax.ShapeDtypeStruct((D_EDGE * _NP,), jnp.float32),
        ],
        scratch_types=[
            pltpu.VMEM((_SCC,), jnp.int32),
            pltpu.VMEM((_SCC,), jnp.float32),
            pltpu.VMEM((_SCC,), jnp.int32),
            pltpu.VMEM((_SCC,), jnp.float32),
            pltpu.VMEM((_NP,), jnp.float32),
            pltpu.SemaphoreType.DMA,
        ],
    )
    def k(ws_h, wr_h, s_h, r_h, sa_h, ra_h,
          idx_a, col_a, idx_b, col_b, acc_v, sem_l):
        cid = lax.axis_index("c")
        sid = lax.axis_index("s")
        zero16 = jnp.zeros((16,), jnp.float32)

        @pl.loop(0, _NP // 16)
        def _(i):
            acc_v[pl.ds(i * 16, 16)] = zero16

        def run(dat_h, i_h, o_h):
            colbase = sid * E

            def start_load(c, idx_v, col_v):
                off = c * _SCC
                pltpu.async_copy(i_h.at[pl.ds(off, _SCC)], idx_v, sem_l)
                pltpu.async_copy(dat_h.at[pl.ds(colbase + off, _SCC)],
                                 col_v, sem_l)

            def wait_load(c, idx_v, col_v):
                off = c * _SCC
                pltpu.make_async_copy(
                    i_h.at[pl.ds(off, _SCC)], idx_v, sem_l).wait()
                pltpu.make_async_copy(
                    dat_h.at[pl.ds(colbase + off, _SCC)], col_v, sem_l).wait()

            def process(idx_v, col_v):
                @pl.loop(0, _SNG // 5)
                def _(g5):
                    for u in range(5):
                        g = g5 * 5 + u
                        iv = idx_v[pl.ds(g * 16, 16)]
                        vv = col_v[pl.ds(g * 16, 16)]
                        plsc.addupdate_scatter(acc_v, [iv], vv)

            start_load(0, idx_a, col_a)

            @pl.loop(0, _SNC // 2)
            def _(q):
                c0 = 2 * q
                wait_load(c0, idx_a, col_a)
                start_load(c0 + 1, idx_b, col_b)
                process(idx_a, col_a)
                wait_load(c0 + 1, idx_b, col_b)

                @pl.when(c0 + 2 < _SNC)
                def _():
                    start_load(c0 + 2, idx_a, col_a)

                process(idx_b, col_b)

            pltpu.sync_copy(acc_v, o_h.at[pl.ds(sid * _NP, _NP)])

        @pl.when(cid == 0)
        def _():
            run(ws_h, s_h, sa_h)

        @pl.when(cid == 1)
        def _():
            run(wr_h, r_h, ra_h)

    return k(wst_f, wrt_f, senders, receivers)


def _node_kernel(n_ref, ra_ref, sa_ref,
                 w1a_ref, w1b_ref, w1c_ref, b1_ref,
                 w2_ref, b2_ref, w3_ref, b3_ref, gn_ref, bgn_ref, out_ref):
    x = n_ref[...]
    h = (_mm(x, w1a_ref[...]) + _mmT(ra_ref[...], w1b_ref[...])
         + _mmT(sa_ref[...], w1c_ref[...]) + b1_ref[...])
    h = h * jax.nn.sigmoid(h)
    h = _mm(h, w2_ref[...]) + b2_ref[...]
    h = h * jax.nn.sigmoid(h)
    h = _mm(h, w3_ref[...]) + b3_ref[...]
    mu = jnp.mean(h, axis=1, keepdims=True)
    d = h - mu
    var = jnp.mean(d * d, axis=1, keepdims=True)
    un = d * jax.lax.rsqrt(var + 1e-5) * gn_ref[...] + bgn_ref[...]
    out_ref[...] = x + un


def _node_call(nodes, recv_agg, sent_agg, Wn1a, Wn1b, Wn1c, bn1,
               Wn2, bn2, Wn3, bn3, gn, bgn):
    rep2 = lambda shape: pl.BlockSpec(shape, lambda i: (0, 0))
    blk = lambda w: pl.BlockSpec((_BN, w), lambda i: (i, 0))
    blkp = lambda w: pl.BlockSpec((_BNP, w), lambda i: (i, 0))
    return pl.pallas_call(
        _node_kernel,
        grid=(_NP // _BNP,),
        in_specs=[
            blkp(D_NODE),
            pl.BlockSpec((D_EDGE, _BNP), lambda i: (0, i)),
            pl.BlockSpec((D_EDGE, _BNP), lambda i: (0, i)),
            rep2((D_NODE, HID)), rep2((D_EDGE, HID)), rep2((D_EDGE, HID)),
            rep2((1, HID)),
            rep2((HID, HID)), rep2((1, HID)),
            rep2((HID, D_NODE)), rep2((1, D_NODE)),
            rep2((1, D_NODE)), rep2((1, D_NODE)),
        ],
        out_specs=blkp(D_NODE),
        out_shape=jax.ShapeDtypeStruct((_NP, D_NODE), jnp.float32),
    )(nodes, recv_agg, sent_agg, Wn1a, Wn1b, Wn1c, bn1,
      Wn2, bn2, Wn3, bn3, gn, bgn)


def kernel(nodes, edges, r, senders, receivers,
           We1, be1, We2, be2, We3, be3, ge, bge,
           Wn1, bn1, Wn2, bn2, Wn3, bn3, gn, bgn,
           Wra, bra, Wsa, bsa):
    We1e = We1[:D_EDGE]
    We1s = We1[D_EDGE:D_EDGE + D_NODE]
    We1r = We1[D_EDGE + D_NODE:]

    ns1, nr1 = _tables_call(nodes, We1s, We1r)
    gs, gr = _gather_call(ns1, nr1, senders, receivers)
    cut = _cut_call(r.reshape(E // 128, 128)).reshape(E, 1)

    new_edges, wst, wrt = _edge_call(
        edges, gs, gr, cut,
        We1e, be1.reshape(1, HID), We2, be2.reshape(1, HID),
        We3, be3.reshape(1, D_EDGE),
        ge.reshape(1, D_EDGE), bge.reshape(1, D_EDGE),
        jnp.tile(Wra, (1, D_EDGE)), bra.reshape(1, 1),
        jnp.tile(Wsa, (1, D_EDGE)), bsa.reshape(1, 1))

    sat_f, rat_f = _scatter_call(wst.reshape(D_EDGE * E), wrt.reshape(D_EDGE * E),
                                 senders, receivers)
    sent_agg = sat_f.reshape(D_EDGE, _NP)
    recv_agg = rat_f.reshape(D_EDGE, _NP)

    Wn1a = Wn1[:D_NODE]
    Wn1b = Wn1[D_NODE:D_NODE + D_EDGE]
    Wn1c = Wn1[D_NODE + D_EDGE:]
    nodes_p = jnp.pad(nodes, ((0, _NP - N), (0, 0)))
    new_nodes_p = _node_call(
        nodes_p, recv_agg, sent_agg, Wn1a, Wn1b, Wn1c,
        bn1.reshape(1, HID), Wn2, bn2.reshape(1, HID),
        Wn3, bn3.reshape(1, D_NODE),
        gn.reshape(1, D_NODE), bgn.reshape(1, D_NODE))

    return (new_nodes_p[:N], new_edges)


# speedup vs baseline: 3.8893x; 1.0299x over previous
import dataclasses
import functools

import jax
import jax.numpy as jnp
from jax import lax
from jax.experimental import pallas as pl
from jax.experimental.pallas import tpu as pltpu
from jax.experimental.pallas import tpu_sc as plsc

N = 10000
E = 320000
D_NODE = 128
D_EDGE = 16
HID = 128
R_MAX = 6.0

_NC = 2
_NS = 16
_NW = _NC * _NS

_GCH = 80
_EPW = E // _NW
_NBUF = 4

_SCC = 20000
_SNC = E // _SCC
_SNG = _SCC // 16

_NP = 10240
_BN = 400
_BNP = 512
_BE = 6400


def _mm(a, b):
    return jax.lax.dot_general(a, b, (((1,), (0,)), ((), ())),
                               preferred_element_type=jnp.float32)


def _mmT(a, b):
    return jax.lax.dot_general(a, b, (((0,), (0,)), ((), ())),
                               preferred_element_type=jnp.float32)


def _tables_kernel(nodes_ref, ws_ref, wr_ref, ns_ref, nr_ref):
    x = nodes_ref[...]
    ns_ref[...] = _mm(x, ws_ref[...])
    nr_ref[...] = _mm(x, wr_ref[...])


def _tables_call(nodes, We1s, We1r):
    return pl.pallas_call(
        _tables_kernel,
        grid=(N // _BN,),
        in_specs=[
            pl.BlockSpec((_BN, D_NODE), lambda i: (i, 0)),
            pl.BlockSpec((D_NODE, HID), lambda i: (0, 0)),
            pl.BlockSpec((D_NODE, HID), lambda i: (0, 0)),
        ],
        out_specs=[
            pl.BlockSpec((_BN, HID), lambda i: (i, 0)),
            pl.BlockSpec((_BN, HID), lambda i: (i, 0)),
        ],
        out_shape=[
            jax.ShapeDtypeStruct((N, HID), jnp.float32),
            jax.ShapeDtypeStruct((N, HID), jnp.float32),
        ],
    )(nodes, We1s, We1r)


def _gather_call(ns1, nr1, senders, receivers):
    mesh = plsc.VectorSubcoreMesh(core_axis_name="c", subcore_axis_name="s")

    @functools.partial(
        pl.kernel,
        mesh=mesh,
        out_type=[
            jax.ShapeDtypeStruct((E, HID), jnp.float32),
            jax.ShapeDtypeStruct((E, HID), jnp.float32),
        ],
        scratch_types=(
            [pltpu.VMEM((_EPW,), jnp.int32)] * 2
            + [pltpu.VMEM((_GCH, HID), jnp.float32)] * (2 * _NBUF)
            + [pltpu.SemaphoreType.DMA] * (_NBUF + 1)
        ),
    )
    def k(ns1_h, nr1_h, s_h, r_h, gs_h, gr_h, is_v, ir_v, *rest):
        bufs = rest[:2 * _NBUF]
        sems = rest[2 * _NBUF:]
        sem_w = sems[_NBUF]
        wid = lax.axis_index("s") * _NC + lax.axis_index("c")
        base = wid * _EPW
        pltpu.sync_copy(s_h.at[pl.ds(base, _EPW)], is_v)
        pltpu.sync_copy(r_h.at[pl.ds(base, _EPW)], ir_v)

        def gather(off, b):
            g1 = pltpu.async_copy(ns1_h.at[is_v.at[pl.ds(off, _GCH)]],
                                  bufs[2 * b], sems[b])
            g2 = pltpu.async_copy(nr1_h.at[ir_v.at[pl.ds(off, _GCH)]],
                                  bufs[2 * b + 1], sems[b])
            return g1, g2

        def write(off, b):
            sl = pl.ds(base + off, _GCH)
            w1 = pltpu.async_copy(bufs[2 * b], gs_h.at[sl], sem_w)
            w2 = pltpu.async_copy(bufs[2 * b + 1], gr_h.at[sl], sem_w)
            return w1, w2

        @pl.loop(0, (_EPW // _GCH) // _NBUF)
        def _(q):
            base_off = q * (_NBUF * _GCH)
            gs_pend = [gather(base_off + b * _GCH, b) for b in range(_NBUF)]
            ws_pend = []
            for b in range(_NBUF):
                g1, g2 = gs_pend[b]
                g1.wait()
                g2.wait()
                ws_pend.append(write(base_off + b * _GCH, b))
            for w1, w2 in ws_pend:
                w1.wait()
                w2.wait()

        tail = ((_EPW // _GCH) // _NBUF) * _NBUF * _GCH
        g1, g2 = gather(tail, 0)
        g1.wait()
        g2.wait()
        w1, w2 = write(tail, 0)
        w1.wait()
        w2.wait()

    return k(ns1, nr1, senders, receivers)


def _cut_kernel(r_ref, out_ref):
    r = r_ref[...]
    x = r * (1.0 / R_MAX)
    x2 = x * x
    x4 = x2 * x2
    x5 = x4 * x
    x6 = x5 * x
    envelope = 1.0 - 15.0 * x4 + 24.0 * x5 - 10.0 * x6
    out_ref[...] = jnp.where(r < R_MAX, envelope, 0.0)


def _cut_call(r2):
    rows = E // 128
    return pl.pallas_call(
        _cut_kernel,
        grid=(1,),
        in_specs=[pl.BlockSpec((rows, 128), lambda i: (0, 0))],
        out_specs=pl.BlockSpec((rows, 128), lambda i: (0, 0)),
        out_shape=jax.ShapeDtypeStruct((rows, 128), jnp.float32),
    )(r2)


def _edge_kernel(e_ref, gs_ref, gr_ref, cut_ref,
                 we1_ref, be1_ref, we2_ref, be2_ref, we3_ref, be3_ref,
                 ge_ref, bge_ref, wra_ref, bra_ref, wsa_ref, bsa_ref,
                 ne_ref, ws_ref, wr_ref):
    e = e_ref[...]
    cut = _mm(cut_ref[...], jnp.ones((1, D_EDGE), dtype=jnp.float32))
    ra = jax.nn.sigmoid(_mm(e, wra_ref[...]) + bra_ref[0, 0]) * cut
    sa = jax.nn.sigmoid(_mm(e, wsa_ref[...]) + bsa_ref[0, 0]) * cut
    h = _mm(e, we1_ref[...]) + gs_ref[...] + gr_ref[...] + be1_ref[...]
    h = h * jax.nn.sigmoid(h)
    h = _mm(h, we2_ref[...]) + be2_ref[...]
    h = h * jax.nn.sigmoid(h)
    h = _mm(h, we3_ref[...]) + be3_ref[...]
    avg16 = jnp.full((D_EDGE, D_EDGE), 1.0 / D_EDGE, dtype=jnp.float32)
    mu = _mm(h, avg16)
    d = h - mu
    var = _mm(d * d, avg16)
    ue = d * jax.lax.rsqrt(var + 1e-5) * ge_ref[...] + bge_ref[...]
    ne_ref[...] = e + ue
    ws_ref[...] = (ue * sa).T
    wr_ref[...] = (ue * ra).T


def _edge_call(edges, gs, gr, cut, We1e, be1, We2, be2, We3, be3,
               ge, bge, wra, bra, wsa, bsa):
    rep2 = lambda shape: pl.BlockSpec(shape, lambda i: (0, 0))
    blk = lambda w: pl.BlockSpec((_BE, w), lambda i: (i, 0))
    return pl.pallas_call(
        _edge_kernel,
        grid=(E // _BE,),
        in_specs=[
            blk(D_EDGE), blk(HID), blk(HID), blk(1),
            rep2((D_EDGE, HID)), rep2((1, HID)),
            rep2((HID, HID)), rep2((1, HID)),
            rep2((HID, D_EDGE)), rep2((1, D_EDGE)),
            rep2((1, D_EDGE)), rep2((1, D_EDGE)),
            rep2((D_EDGE, D_EDGE)), rep2((1, 1)),
            rep2((D_EDGE, D_EDGE)), rep2((1, 1)),
        ],
        out_specs=[
            blk(D_EDGE),
            pl.BlockSpec((D_EDGE, _BE), lambda i: (0, i)),
            pl.BlockSpec((D_EDGE, _BE), lambda i: (0, i)),
        ],
        out_shape=[
            jax.ShapeDtypeStruct((E, D_EDGE), jnp.float32),
            jax.ShapeDtypeStruct((D_EDGE, E), jnp.float32),
            jax.ShapeDtypeStruct((D_EDGE, E), jnp.float32),
        ],
    )(edges, gs, gr, cut, We1e, be1, We2, be2, We3, be3,
      ge, bge, wra, bra, wsa, bsa)


def _scatter_call(wst_f, wrt_f, senders, receivers):
    mesh = plsc.VectorSubcoreMesh(core_axis_name="c", subcore_axis_name="s")
    cp = pltpu.CompilerParams()
    if "needs_layout_passes" in pltpu.CompilerParams.__dataclass_fields__:
        cp = dataclasses.replace(cp, needs_layout_passes=False)

    @functools.partial(
        pl.kernel,
        mesh=mesh,
        compiler_params=cp,
        out_type=[
            jax.ShapeDtypeStruct((D_EDGE * _NP,), jnp.float32),
            jax.ShapeDtypeStruct((D_EDGE * _NP,), jnp.float32),
        ],
        scratch_types=[
            pltpu.VMEM((_SCC,), jnp.int32),
            pltpu.VMEM((_SCC,), jnp.float32),
            pltpu.VMEM((_SCC,), jnp.int32),
            pltpu.VMEM((_SCC,), jnp.float32),
            pltpu.VMEM((_NP,), jnp.float32),
            pltpu.SemaphoreType.DMA,
        ],
    )
    def k(ws_h, wr_h, s_h, r_h, sa_h, ra_h,
          idx_a, col_a, idx_b, col_b, acc_v, sem_l):
        cid = lax.axis_index("c")
        sid = lax.axis_index("s")
        zero16 = jnp.zeros((16,), jnp.float32)

        @pl.loop(0, _NP // 16)
        def _(i):
            acc_v[pl.ds(i * 16, 16)] = zero16

        def run(dat_h, i_h, o_h):
            colbase = sid * E

            def start_load(c, idx_v, col_v):
                off = c * _SCC
                pltpu.async_copy(i_h.at[pl.ds(off, _SCC)], idx_v, sem_l)
                pltpu.async_copy(dat_h.at[pl.ds(colbase + off, _SCC)],
                                 col_v, sem_l)

            def wait_load(c, idx_v, col_v):
                off = c * _SCC
                pltpu.make_async_copy(
                    i_h.at[pl.ds(off, _SCC)], idx_v, sem_l).wait()
                pltpu.make_async_copy(
                    dat_h.at[pl.ds(colbase + off, _SCC)], col_v, sem_l).wait()

            def process(idx_v, col_v):
                @pl.loop(0, _SNG // 5)
                def _(g5):
                    for u in range(5):
                        g = g5 * 5 + u
                        iv = idx_v[pl.ds(g * 16, 16)]
                        vv = col_v[pl.ds(g * 16, 16)]
                        plsc.addupdate_scatter(acc_v, [iv], vv)

            start_load(0, idx_a, col_a)

            @pl.loop(0, _SNC // 2)
            def _(q):
                c0 = 2 * q
                wait_load(c0, idx_a, col_a)
                start_load(c0 + 1, idx_b, col_b)
                process(idx_a, col_a)
                wait_load(c0 + 1, idx_b, col_b)

                @pl.when(c0 + 2 < _SNC)
                def _():
                    start_load(c0 + 2, idx_a, col_a)

                process(idx_b, col_b)

            pltpu.sync_copy(acc_v, o_h.at[pl.ds(sid * _NP, _NP)])

        @pl.when(cid == 0)
        def _():
            run(ws_h, s_h, sa_h)

        @pl.when(cid == 1)
        def _():
            run(wr_h, r_h, ra_h)

    return k(wst_f, wrt_f, senders, receivers)


def _node_kernel(n_ref, ra_ref, sa_ref,
                 w1a_ref, w1b_ref, w1c_ref, b1_ref,
                 w2_ref, b2_ref, w3_ref, b3_ref, gn_ref, bgn_ref, out_ref):
    x = n_ref[...]
    h = (_mm(x, w1a_ref[...]) + _mmT(ra_ref[...], w1b_ref[...])
         + _mmT(sa_ref[...], w1c_ref[...]) + b1_ref[...])
    h = h * jax.nn.sigmoid(h)
    h = _mm(h, w2_ref[...]) + b2_ref[...]
    h = h * jax.nn.sigmoid(h)
    h = _mm(h, w3_ref[...]) + b3_ref[...]
    mu = jnp.mean(h, axis=1, keepdims=True)
    d = h - mu
    var = jnp.mean(d * d, axis=1, keepdims=True)
    un = d * jax.lax.rsqrt(var + 1e-5) * gn_ref[...] + bgn_ref[...]
    out_ref[...] = x + un


def _node_call(nodes, recv_agg, sent_agg, Wn1a, Wn1b, Wn1c, bn1,
               Wn2, bn2, Wn3, bn3, gn, bgn):
    rep2 = lambda shape: pl.BlockSpec(shape, lambda i: (0, 0))
    blk = lambda w: pl.BlockSpec((_BN, w), lambda i: (i, 0))
    blkp = lambda w: pl.BlockSpec((_BNP, w), lambda i: (i, 0))
    return pl.pallas_call(
        _node_kernel,
        grid=(_NP // _BNP,),
        in_specs=[
            blkp(D_NODE),
            pl.BlockSpec((D_EDGE, _BNP), lambda i: (0, i)),
            pl.BlockSpec((D_EDGE, _BNP), lambda i: (0, i)),
            rep2((D_NODE, HID)), rep2((D_EDGE, HID)), rep2((D_EDGE, HID)),
            rep2((1, HID)),
            rep2((HID, HID)), rep2((1, HID)),
            rep2((HID, D_NODE)), rep2((1, D_NODE)),
            rep2((1, D_NODE)), rep2((1, D_NODE)),
        ],
        out_specs=blkp(D_NODE),
        out_shape=jax.ShapeDtypeStruct((_NP, D_NODE), jnp.float32),
    )(nodes, recv_agg, sent_agg, Wn1a, Wn1b, Wn1c, bn1,
      Wn2, bn2, Wn3, bn3, gn, bgn)


def kernel(nodes, edges, r, senders, receivers,
           We1, be1, We2, be2, We3, be3, ge, bge,
           Wn1, bn1, Wn2, bn2, Wn3, bn3, gn, bgn,
           Wra, bra, Wsa, bsa):
    We1e = We1[:D_EDGE]
    We1s = We1[D_EDGE:D_EDGE + D_NODE]
    We1r = We1[D_EDGE + D_NODE:]

    ns1, nr1 = _tables_call(nodes, We1s, We1r)
    gs, gr = _gather_call(ns1, nr1, senders, receivers)
    cut = _cut_call(r.reshape(E // 128, 128)).reshape(E, 1)

    new_edges, wst, wrt = _edge_call(
        edges, gs, gr, cut,
        We1e, be1.reshape(1, HID), We2, be2.reshape(1, HID),
        We3, be3.reshape(1, D_EDGE),
        ge.reshape(1, D_EDGE), bge.reshape(1, D_EDGE),
        jnp.tile(Wra, (1, D_EDGE)), bra.reshape(1, 1),
        jnp.tile(Wsa, (1, D_EDGE)), bsa.reshape(1, 1))

    sat_f, rat_f = _scatter_call(wst.reshape(D_EDGE * E), wrt.reshape(D_EDGE * E),
                                 senders, receivers)
    sent_agg = sat_f.reshape(D_EDGE, _NP)
    recv_agg = rat_f.reshape(D_EDGE, _NP)

    Wn1a = Wn1[:D_NODE]
    Wn1b = Wn1[D_NODE:D_NODE + D_EDGE]
    Wn1c = Wn1[D_NODE + D_EDGE:]
    nodes_p = jnp.pad(nodes, ((0, _NP - N), (0, 0)))
    new_nodes_p = _node_call(
        nodes_p, recv_agg, sent_agg, Wn1a, Wn1b, Wn1c,
        bn1.reshape(1, HID), Wn2, bn2.reshape(1, HID),
        Wn3, bn3.reshape(1, D_NODE),
        gn.reshape(1, D_NODE), bgn.reshape(1, D_NODE))

    return (new_nodes_p[:N], new_edges)
